# baseline (device time: 127085 ns/iter reference)
import jax
import jax.numpy as jnp
from jax import lax
from jax.experimental import pallas as pl
from jax.experimental.pallas import tpu as pltpu

N_DEV = 16
B = 2
SQ = 128
SKV = 128
HQ = 4
DH = 64
BLK = 64
D_MODEL = 512
D_QK = HQ * DH
SKV_ALL = N_DEV * SKV


def kernel(x, Wq, K_ext, V_ext, Wo):
    k2 = K_ext.reshape(B, SKV, D_QK)
    v2 = V_ext.reshape(B, SKV, D_QK)

    def body(x_ref, wq_ref, k_ref, v_ref, wo_ref, out_ref,
             gk_ref, gv_ref, send_k, recv_k, send_v, recv_v):
        my = lax.axis_index("i")
        left = (my - 1) % N_DEV
        right = (my + 1) % N_DEV

        barrier = pltpu.get_barrier_semaphore()
        for nbr in (left, right):
            pl.semaphore_signal(barrier, inc=1, device_id=(nbr,),
                                device_id_type=pl.DeviceIdType.MESH)
        pl.semaphore_wait(barrier, 2)

        gk_ref[pl.ds(my, 1)] = k_ref[...][None]
        gv_ref[pl.ds(my, 1)] = v_ref[...][None]

        for h in range(N_DEV - 1):
            slot = (my - h) % N_DEV
            rk = pltpu.make_async_remote_copy(
                src_ref=gk_ref.at[slot], dst_ref=gk_ref.at[slot],
                send_sem=send_k.at[h], recv_sem=recv_k.at[h],
                device_id=(right,), device_id_type=pl.DeviceIdType.MESH)
            rv = pltpu.make_async_remote_copy(
                src_ref=gv_ref.at[slot], dst_ref=gv_ref.at[slot],
                send_sem=send_v.at[h], recv_sem=recv_v.at[h],
                device_id=(right,), device_id_type=pl.DeviceIdType.MESH)
            rk.start()
            rv.start()
            rk.wait()
            rv.wait()

        i_idx = lax.broadcasted_iota(jnp.int32, (SQ, SKV_ALL), 0)
        j_idx = lax.broadcasted_iota(jnp.int32, (SQ, SKV_ALL), 1)
        qb = (my * SQ + i_idx) // BLK
        kb = j_idx // BLK
        vis = (qb == kb) | (kb == 0) | ((qb + kb) % 3 == 0)

        for b in range(B):
            q_b = jnp.dot(x_ref[b], wq_ref[...],
                          preferred_element_type=jnp.float32)
            k_all = gk_ref[:, b].reshape(SKV_ALL, D_QK)
            v_all = gv_ref[:, b].reshape(SKV_ALL, D_QK)
            ctxs = []
            for hh in range(HQ):
                q_h = q_b[:, hh * DH:(hh + 1) * DH]
                k_h = k_all[:, hh * DH:(hh + 1) * DH]
                v_h = v_all[:, hh * DH:(hh + 1) * DH]
                s = lax.dot_general(
                    q_h, k_h, (((1,), (1,)), ((), ())),
                    preferred_element_type=jnp.float32) * 0.125
                s = jnp.where(vis, s, -1e9)
                m = jnp.max(s, axis=1, keepdims=True)
                w = jnp.exp(s - m)
                w = w / jnp.sum(w, axis=1, keepdims=True)
                ctxs.append(jnp.dot(w, v_h,
                                    preferred_element_type=jnp.float32))
            ctx = jnp.concatenate(ctxs, axis=1)
            out_ref[b] = jnp.dot(ctx, wo_ref[...],
                                 preferred_element_type=jnp.float32)

    return pl.pallas_call(
        body,
        out_shape=jax.ShapeDtypeStruct((B, SQ, D_MODEL), jnp.float32),
        in_specs=[pl.BlockSpec(memory_space=pltpu.VMEM)] * 5,
        out_specs=pl.BlockSpec(memory_space=pltpu.VMEM),
        scratch_shapes=[
            pltpu.VMEM((N_DEV, B, SKV, D_QK), jnp.float32),
            pltpu.VMEM((N_DEV, B, SKV, D_QK), jnp.float32),
            pltpu.SemaphoreType.DMA((N_DEV - 1,)),
            pltpu.SemaphoreType.DMA((N_DEV - 1,)),
            pltpu.SemaphoreType.DMA((N_DEV - 1,)),
            pltpu.SemaphoreType.DMA((N_DEV - 1,)),
        ],
        compiler_params=pltpu.CompilerParams(collective_id=0),
    )(x, Wq, k2, v2, Wo)


# device time: 87297 ns/iter; 1.4558x vs baseline; 1.4558x over previous
import jax
import jax.numpy as jnp
from jax import lax
from jax.experimental import pallas as pl
from jax.experimental.pallas import tpu as pltpu

N_DEV = 16
B = 2
SQ = 128
SKV = 128
HQ = 4
DH = 64
BLK = 64
D_MODEL = 512
D_QK = HQ * DH
SKV_ALL = N_DEV * SKV


def kernel(x, Wq, K_ext, V_ext, Wo):
    k2 = K_ext.reshape(B, SKV, D_QK)
    v2 = V_ext.reshape(B, SKV, D_QK)

    FWD_HOPS = 8
    BWD_HOPS = 7

    def body(x_ref, wq_ref, k_ref, v_ref, wo_ref, out_ref,
             gk_ref, gv_ref,
             fsend_k, frecv_k, fsend_v, frecv_v,
             bsend_k, brecv_k, bsend_v, brecv_v):
        my = lax.axis_index("i")
        left = (my - 1) % N_DEV
        right = (my + 1) % N_DEV

        barrier = pltpu.get_barrier_semaphore()
        for nbr in (left, right):
            pl.semaphore_signal(barrier, inc=1, device_id=(nbr,),
                                device_id_type=pl.DeviceIdType.MESH)
        pl.semaphore_wait(barrier, 2)

        gk_ref[pl.ds(my, 1)] = k_ref[...][None]
        gv_ref[pl.ds(my, 1)] = v_ref[...][None]

        def mk(slot, sems_s, sems_r, h, gref, dev):
            return pltpu.make_async_remote_copy(
                src_ref=gref.at[slot], dst_ref=gref.at[slot],
                send_sem=sems_s.at[h], recv_sem=sems_r.at[h],
                device_id=(dev,), device_id_type=pl.DeviceIdType.MESH)

        for h in range(FWD_HOPS):
            fslot = (my - h) % N_DEV
            fk = mk(fslot, fsend_k, frecv_k, h, gk_ref, right)
            fv = mk(fslot, fsend_v, frecv_v, h, gv_ref, right)
            fk.start()
            fv.start()
            if h < BWD_HOPS:
                bslot = (my + h) % N_DEV
                bk = mk(bslot, bsend_k, brecv_k, h, gk_ref, left)
                bv = mk(bslot, bsend_v, brecv_v, h, gv_ref, left)
                bk.start()
                bv.start()
                bk.wait()
                bv.wait()
            fk.wait()
            fv.wait()

        i_idx = lax.broadcasted_iota(jnp.int32, (SQ, SKV_ALL), 0)
        j_idx = lax.broadcasted_iota(jnp.int32, (SQ, SKV_ALL), 1)
        qb = (my * SQ + i_idx) // BLK
        kb = j_idx // BLK
        vis = (qb == kb) | (kb == 0) | ((qb + kb) % 3 == 0)

        for b in range(B):
            q_b = jnp.dot(x_ref[b], wq_ref[...],
                          preferred_element_type=jnp.float32)
            k_all = gk_ref[:, b].reshape(SKV_ALL, D_QK)
            v_all = gv_ref[:, b].reshape(SKV_ALL, D_QK)
            ctxs = []
            for hh in range(HQ):
                q_h = q_b[:, hh * DH:(hh + 1) * DH]
                k_h = k_all[:, hh * DH:(hh + 1) * DH]
                v_h = v_all[:, hh * DH:(hh + 1) * DH]
                s = lax.dot_general(
                    q_h, k_h, (((1,), (1,)), ((), ())),
                    preferred_element_type=jnp.float32) * 0.125
                s = jnp.where(vis, s, -1e9)
                m = jnp.max(s, axis=1, keepdims=True)
                w = jnp.exp(s - m)
                w = w / jnp.sum(w, axis=1, keepdims=True)
                ctxs.append(jnp.dot(w, v_h,
                                    preferred_element_type=jnp.float32))
            ctx = jnp.concatenate(ctxs, axis=1)
            out_ref[b] = jnp.dot(ctx, wo_ref[...],
                                 preferred_element_type=jnp.float32)

    return pl.pallas_call(
        body,
        out_shape=jax.ShapeDtypeStruct((B, SQ, D_MODEL), jnp.float32),
        in_specs=[pl.BlockSpec(memory_space=pltpu.VMEM)] * 5,
        out_specs=pl.BlockSpec(memory_space=pltpu.VMEM),
        scratch_shapes=[
            pltpu.VMEM((N_DEV, B, SKV, D_QK), jnp.float32),
            pltpu.VMEM((N_DEV, B, SKV, D_QK), jnp.float32),
            pltpu.SemaphoreType.DMA((8,)),
            pltpu.SemaphoreType.DMA((8,)),
            pltpu.SemaphoreType.DMA((8,)),
            pltpu.SemaphoreType.DMA((8,)),
            pltpu.SemaphoreType.DMA((7,)),
            pltpu.SemaphoreType.DMA((7,)),
            pltpu.SemaphoreType.DMA((7,)),
            pltpu.SemaphoreType.DMA((7,)),
        ],
        compiler_params=pltpu.CompilerParams(collective_id=0),
    )(x, Wq, k2, v2, Wo)


# device time: 87111 ns/iter; 1.4589x vs baseline; 1.0021x over previous
import jax
import jax.numpy as jnp
from jax import lax
from jax.experimental import pallas as pl
from jax.experimental.pallas import tpu as pltpu

N_DEV = 16
B = 2
SQ = 128
SKV = 128
HQ = 4
DH = 64
BLK = 64
D_MODEL = 512
D_QK = HQ * DH
SKV_ALL = N_DEV * SKV


def kernel(x, Wq, K_ext, V_ext, Wo):
    k2 = K_ext.reshape(B, SKV, D_QK)
    v2 = V_ext.reshape(B, SKV, D_QK)

    FWD_HOPS = 8
    BWD_HOPS = 7

    def _ring_to_mesh(r):
        s = r // 4
        zz = r % 4
        z = jnp.where(s % 2 == 0, zz, 3 - zz)
        x = ((s + 1) // 2) % 2
        y = s // 2
        return 8 * x + 4 * y + z

    def _mesh_to_ring(m):
        x = m // 8
        y = (m % 8) // 4
        z = m % 4
        s = 2 * y + (x + y) % 2
        zz = jnp.where(s % 2 == 0, z, 3 - z)
        return 4 * s + zz

    def body(x_ref, wq_ref, k_ref, v_ref, wo_ref, out_ref,
             gk_ref, gv_ref,
             fsend_k, frecv_k, fsend_v, frecv_v,
             bsend_k, brecv_k, bsend_v, brecv_v):
        my = lax.axis_index("i")
        rpos = _mesh_to_ring(my)
        left = _ring_to_mesh((rpos - 1) % N_DEV)
        right = _ring_to_mesh((rpos + 1) % N_DEV)

        barrier = pltpu.get_barrier_semaphore()
        for nbr in (left, right):
            pl.semaphore_signal(barrier, inc=1, device_id=(nbr,),
                                device_id_type=pl.DeviceIdType.MESH)
        pl.semaphore_wait(barrier, 2)

        gk_ref[pl.ds(my, 1)] = k_ref[...][None]
        gv_ref[pl.ds(my, 1)] = v_ref[...][None]

        def mk(slot, sems_s, sems_r, h, gref, dev):
            return pltpu.make_async_remote_copy(
                src_ref=gref.at[slot], dst_ref=gref.at[slot],
                send_sem=sems_s.at[h], recv_sem=sems_r.at[h],
                device_id=(dev,), device_id_type=pl.DeviceIdType.MESH)

        fks, fvs, bks, bvs = [], [], [], []
        for h in range(FWD_HOPS):
            fslot = _ring_to_mesh((rpos - h) % N_DEV)
            fks.append(mk(fslot, fsend_k, frecv_k, h, gk_ref, right))
            fvs.append(mk(fslot, fsend_v, frecv_v, h, gv_ref, right))
        for h in range(BWD_HOPS):
            bslot = _ring_to_mesh((rpos + h) % N_DEV)
            bks.append(mk(bslot, bsend_k, brecv_k, h, gk_ref, left))
            bvs.append(mk(bslot, bsend_v, brecv_v, h, gv_ref, left))

        fks[0].start()
        bks[0].start()
        fvs[0].start()
        bvs[0].start()
        for h in range(1, FWD_HOPS):
            fks[h - 1].wait_recv()
            fks[h].start()
            if h < BWD_HOPS:
                bks[h - 1].wait_recv()
                bks[h].start()
            fvs[h - 1].wait_recv()
            fvs[h].start()
            if h < BWD_HOPS:
                bvs[h - 1].wait_recv()
                bvs[h].start()
        fks[-1].wait_recv()
        fvs[-1].wait_recv()
        bks[-1].wait_recv()
        bvs[-1].wait_recv()
        for d in fks + fvs + bks + bvs:
            d.wait_send()

        i_idx = lax.broadcasted_iota(jnp.int32, (SQ, SKV_ALL), 0)
        j_idx = lax.broadcasted_iota(jnp.int32, (SQ, SKV_ALL), 1)
        qb = (my * SQ + i_idx) // BLK
        kb = j_idx // BLK
        vis = (qb == kb) | (kb == 0) | ((qb + kb) % 3 == 0)

        for b in range(B):
            q_b = jnp.dot(x_ref[b], wq_ref[...],
                          preferred_element_type=jnp.float32)
            k_all = gk_ref[:, b].reshape(SKV_ALL, D_QK)
            v_all = gv_ref[:, b].reshape(SKV_ALL, D_QK)
            ctxs = []
            for hh in range(HQ):
                q_h = q_b[:, hh * DH:(hh + 1) * DH]
                k_h = k_all[:, hh * DH:(hh + 1) * DH]
                v_h = v_all[:, hh * DH:(hh + 1) * DH]
                s = lax.dot_general(
                    q_h, k_h, (((1,), (1,)), ((), ())),
                    preferred_element_type=jnp.float32) * 0.125
                s = jnp.where(vis, s, -1e9)
                m = jnp.max(s, axis=1, keepdims=True)
                w = jnp.exp(s - m)
                w = w / jnp.sum(w, axis=1, keepdims=True)
                ctxs.append(jnp.dot(w, v_h,
                                    preferred_element_type=jnp.float32))
            ctx = jnp.concatenate(ctxs, axis=1)
            out_ref[b] = jnp.dot(ctx, wo_ref[...],
                                 preferred_element_type=jnp.float32)

    return pl.pallas_call(
        body,
        out_shape=jax.ShapeDtypeStruct((B, SQ, D_MODEL), jnp.float32),
        in_specs=[pl.BlockSpec(memory_space=pltpu.VMEM)] * 5,
        out_specs=pl.BlockSpec(memory_space=pltpu.VMEM),
        scratch_shapes=[
            pltpu.VMEM((N_DEV, B, SKV, D_QK), jnp.float32),
            pltpu.VMEM((N_DEV, B, SKV, D_QK), jnp.float32),
            pltpu.SemaphoreType.DMA((8,)),
            pltpu.SemaphoreType.DMA((8,)),
            pltpu.SemaphoreType.DMA((8,)),
            pltpu.SemaphoreType.DMA((8,)),
            pltpu.SemaphoreType.DMA((7,)),
            pltpu.SemaphoreType.DMA((7,)),
            pltpu.SemaphoreType.DMA((7,)),
            pltpu.SemaphoreType.DMA((7,)),
        ],
        compiler_params=pltpu.CompilerParams(collective_id=0),
    )(x, Wq, k2, v2, Wo)


# device time: 68852 ns/iter; 1.8458x vs baseline; 1.2652x over previous
import jax
import jax.numpy as jnp
from jax import lax
from jax.experimental import pallas as pl
from jax.experimental.pallas import tpu as pltpu

N_DEV = 16
B = 2
SQ = 128
SKV = 128
HQ = 4
DH = 64
BLK = 64
D_MODEL = 512
D_QK = HQ * DH
D_KV2 = 2 * D_QK
SKV_ALL = N_DEV * SKV


def kernel(x, Wq, K_ext, V_ext, Wo):
    c = jnp.concatenate(
        [K_ext.reshape(B, SKV, D_QK), V_ext.reshape(B, SKV, D_QK)], axis=-1)

    def body(x_ref, wq_ref, c_ref, wo_ref, out_ref,
             g_ref, send_sems, recv_sems):
        my = lax.axis_index("i")

        def needs(s, d):
            t = (2 * d + 2 * s) % 3
            n0 = (t != 1) | (s == 0)
            n1 = t != 0
            return n0, n1

        g_ref[pl.ds(my, 1)] = c_ref[...][None]
        for o in range(1, N_DEV):
            s = (my - o) % N_DEV
            n0, n1 = needs(s, my)

            @pl.when(n0 != n1)
            def _(s=s, n0=n0):
                goff = jnp.where(n0, BLK, 0)
                g_ref[pl.ds(s, 1), :, pl.ds(goff, BLK),
                      pl.ds(D_QK, D_QK)] = jnp.zeros(
                          (1, B, BLK, D_QK), jnp.float32)

        barrier = pltpu.get_barrier_semaphore()
        for o in range(1, N_DEV):
            pl.semaphore_signal(barrier, inc=1, device_id=((my + o) % N_DEV,),
                                device_id_type=pl.DeviceIdType.MESH)
        pl.semaphore_wait(barrier, N_DEV - 1)

        def for_each_send(action):
            for o in range(1, N_DEV):
                d = (my + o) % N_DEV
                n0, n1 = needs(my, d)
                roff = jnp.where(n0, 0, BLK)

                @pl.when(n0 & n1)
                def _(d=d, o=o):
                    action(pltpu.make_async_remote_copy(
                        src_ref=c_ref, dst_ref=g_ref.at[my],
                        send_sem=send_sems.at[o - 1],
                        recv_sem=recv_sems.at[o - 1],
                        device_id=(d,),
                        device_id_type=pl.DeviceIdType.MESH))

                @pl.when(n0 != n1)
                def _(d=d, o=o, roff=roff):
                    action(pltpu.make_async_remote_copy(
                        src_ref=c_ref.at[:, pl.ds(roff, BLK), :],
                        dst_ref=g_ref.at[my, :, pl.ds(roff, BLK), :],
                        send_sem=send_sems.at[o - 1],
                        recv_sem=recv_sems.at[o - 1],
                        device_id=(d,),
                        device_id_type=pl.DeviceIdType.MESH))

        for_each_send(lambda r: r.start())

        i_idx = lax.broadcasted_iota(jnp.int32, (SQ, SKV_ALL), 0)
        j_idx = lax.broadcasted_iota(jnp.int32, (SQ, SKV_ALL), 1)
        qb = (my * SQ + i_idx) // BLK
        kb = j_idx // BLK
        vis = (qb == kb) | (kb == 0) | ((qb + kb) % 3 == 0)
        q_all = [jnp.dot(x_ref[b], wq_ref[...],
                         preferred_element_type=jnp.float32)
                 for b in range(B)]

        for o in range(1, N_DEV):
            s = (my - o) % N_DEV
            n0, n1 = needs(s, my)
            roff = jnp.where(n0, 0, BLK)

            @pl.when(n0 & n1)
            def _(s=s, o=o):
                pltpu.make_async_remote_copy(
                    src_ref=c_ref, dst_ref=g_ref.at[s],
                    send_sem=send_sems.at[o - 1],
                    recv_sem=recv_sems.at[o - 1],
                    device_id=(s,),
                    device_id_type=pl.DeviceIdType.MESH).wait_recv()

            @pl.when(n0 != n1)
            def _(s=s, o=o, roff=roff):
                pltpu.make_async_remote_copy(
                    src_ref=c_ref.at[:, pl.ds(roff, BLK), :],
                    dst_ref=g_ref.at[s, :, pl.ds(roff, BLK), :],
                    send_sem=send_sems.at[o - 1],
                    recv_sem=recv_sems.at[o - 1],
                    device_id=(s,),
                    device_id_type=pl.DeviceIdType.MESH).wait_recv()

        for b in range(B):
            k_all = g_ref[:, b, :, 0:D_QK].reshape(SKV_ALL, D_QK)
            v_all = g_ref[:, b, :, D_QK:D_KV2].reshape(SKV_ALL, D_QK)
            ctxs = []
            for hh in range(HQ):
                q_h = q_all[b][:, hh * DH:(hh + 1) * DH]
                k_h = k_all[:, hh * DH:(hh + 1) * DH]
                v_h = v_all[:, hh * DH:(hh + 1) * DH]
                s_ = lax.dot_general(
                    q_h, k_h, (((1,), (1,)), ((), ())),
                    preferred_element_type=jnp.float32) * 0.125
                s_ = jnp.where(vis, s_, -1e9)
                m = jnp.max(s_, axis=1, keepdims=True)
                w = jnp.exp(s_ - m)
                w = w / jnp.sum(w, axis=1, keepdims=True)
                ctxs.append(jnp.dot(w, v_h,
                                    preferred_element_type=jnp.float32))
            ctx = jnp.concatenate(ctxs, axis=1)
            out_ref[b] = jnp.dot(ctx, wo_ref[...],
                                 preferred_element_type=jnp.float32)

        for_each_send(lambda r: r.wait_send())

    return pl.pallas_call(
        body,
        out_shape=jax.ShapeDtypeStruct((B, SQ, D_MODEL), jnp.float32),
        in_specs=[pl.BlockSpec(memory_space=pltpu.VMEM)] * 4,
        out_specs=pl.BlockSpec(memory_space=pltpu.VMEM),
        scratch_shapes=[
            pltpu.VMEM((N_DEV, B, SKV, D_KV2), jnp.float32),
            pltpu.SemaphoreType.DMA((N_DEV - 1,)),
            pltpu.SemaphoreType.DMA((N_DEV - 1,)),
        ],
        compiler_params=pltpu.CompilerParams(collective_id=0),
    )(x, Wq, c, Wo)


# device time: 42091 ns/iter; 3.0193x vs baseline; 1.6358x over previous
import jax
import jax.numpy as jnp
from jax import lax
from jax.experimental import pallas as pl
from jax.experimental.pallas import tpu as pltpu

N_DEV = 16
B = 2
SQ = 128
SKV = 128
HQ = 4
DH = 64
BLK = 64
D_MODEL = 512
D_QK = HQ * DH
D_KV2 = 2 * D_QK
SKV_ALL = N_DEV * SKV


def kernel(x, Wq, K_ext, V_ext, Wo):
    c = jnp.concatenate(
        [K_ext.reshape(B, SKV, D_QK), V_ext.reshape(B, SKV, D_QK)],
        axis=-1).astype(jnp.bfloat16)

    def body(x_ref, wq_ref, c_ref, wo_ref, out_ref,
             g_ref, send_sems, recv_sems):
        my = lax.axis_index("i")

        def needs(s, d):
            t = (2 * d + 2 * s) % 3
            n0 = (t != 1) | (s == 0)
            n1 = t != 0
            return n0, n1

        g_ref[pl.ds(my, 1)] = c_ref[...][None]
        for o in range(1, N_DEV):
            s = (my - o) % N_DEV
            n0, n1 = needs(s, my)

            @pl.when(n0 != n1)
            def _(s=s, n0=n0):
                goff = jnp.where(n0, BLK, 0)
                g_ref[pl.ds(s, 1), :, pl.ds(goff, BLK),
                      pl.ds(D_QK, D_QK)] = jnp.zeros(
                          (1, B, BLK, D_QK), jnp.bfloat16)

        barrier = pltpu.get_barrier_semaphore()
        for o in range(1, N_DEV):
            pl.semaphore_signal(barrier, inc=1, device_id=((my + o) % N_DEV,),
                                device_id_type=pl.DeviceIdType.MESH)
        pl.semaphore_wait(barrier, N_DEV - 1)

        def for_each_send(action):
            for o in range(1, N_DEV):
                d = (my + o) % N_DEV
                n0, n1 = needs(my, d)
                roff = jnp.where(n0, 0, BLK)

                @pl.when(n0 & n1)
                def _(d=d, o=o):
                    action(pltpu.make_async_remote_copy(
                        src_ref=c_ref, dst_ref=g_ref.at[my],
                        send_sem=send_sems.at[o - 1],
                        recv_sem=recv_sems.at[o - 1],
                        device_id=(d,),
                        device_id_type=pl.DeviceIdType.MESH))

                @pl.when(n0 != n1)
                def _(d=d, o=o, roff=roff):
                    action(pltpu.make_async_remote_copy(
                        src_ref=c_ref.at[:, pl.ds(roff, BLK), :],
                        dst_ref=g_ref.at[my, :, pl.ds(roff, BLK), :],
                        send_sem=send_sems.at[o - 1],
                        recv_sem=recv_sems.at[o - 1],
                        device_id=(d,),
                        device_id_type=pl.DeviceIdType.MESH))

        for_each_send(lambda r: r.start())

        i_idx = lax.broadcasted_iota(jnp.int32, (SQ, SKV_ALL), 0)
        j_idx = lax.broadcasted_iota(jnp.int32, (SQ, SKV_ALL), 1)
        qb = (my * SQ + i_idx) // BLK
        kb = j_idx // BLK
        vis = (qb == kb) | (kb == 0) | ((qb + kb) % 3 == 0)
        q_all = [jnp.dot(x_ref[b], wq_ref[...],
                         preferred_element_type=jnp.float32)
                 for b in range(B)]

        for o in range(1, N_DEV):
            s = (my - o) % N_DEV
            n0, n1 = needs(s, my)
            roff = jnp.where(n0, 0, BLK)

            @pl.when(n0 & n1)
            def _(s=s, o=o):
                pltpu.make_async_remote_copy(
                    src_ref=c_ref, dst_ref=g_ref.at[s],
                    send_sem=send_sems.at[o - 1],
                    recv_sem=recv_sems.at[o - 1],
                    device_id=(s,),
                    device_id_type=pl.DeviceIdType.MESH).wait_recv()

            @pl.when(n0 != n1)
            def _(s=s, o=o, roff=roff):
                pltpu.make_async_remote_copy(
                    src_ref=c_ref.at[:, pl.ds(roff, BLK), :],
                    dst_ref=g_ref.at[s, :, pl.ds(roff, BLK), :],
                    send_sem=send_sems.at[o - 1],
                    recv_sem=recv_sems.at[o - 1],
                    device_id=(s,),
                    device_id_type=pl.DeviceIdType.MESH).wait_recv()

        for b in range(B):
            k_all = g_ref[:, b, :, 0:D_QK].reshape(SKV_ALL, D_QK)
            v_all = g_ref[:, b, :, D_QK:D_KV2].reshape(SKV_ALL, D_QK)
            ctxs = []
            for hh in range(HQ):
                q_h = q_all[b][:, hh * DH:(hh + 1) * DH].astype(jnp.bfloat16)
                k_h = k_all[:, hh * DH:(hh + 1) * DH]
                v_h = v_all[:, hh * DH:(hh + 1) * DH]
                s_ = lax.dot_general(
                    q_h, k_h, (((1,), (1,)), ((), ())),
                    preferred_element_type=jnp.float32) * 0.125
                s_ = jnp.where(vis, s_, -1e9)
                m = jnp.max(s_, axis=1, keepdims=True)
                w = jnp.exp(s_ - m)
                w = (w / jnp.sum(w, axis=1, keepdims=True)).astype(jnp.bfloat16)
                ctxs.append(jnp.dot(w, v_h,
                                    preferred_element_type=jnp.float32))
            ctx = jnp.concatenate(ctxs, axis=1)
            out_ref[b] = jnp.dot(ctx, wo_ref[...],
                                 preferred_element_type=jnp.float32)

        for_each_send(lambda r: r.wait_send())

    return pl.pallas_call(
        body,
        out_shape=jax.ShapeDtypeStruct((B, SQ, D_MODEL), jnp.float32),
        in_specs=[pl.BlockSpec(memory_space=pltpu.VMEM)] * 4,
        out_specs=pl.BlockSpec(memory_space=pltpu.VMEM),
        scratch_shapes=[
            pltpu.VMEM((N_DEV, B, SKV, D_KV2), jnp.bfloat16),
            pltpu.SemaphoreType.DMA((N_DEV - 1,)),
            pltpu.SemaphoreType.DMA((N_DEV - 1,)),
        ],
        compiler_params=pltpu.CompilerParams(collective_id=0),
    )(x, Wq, c, Wo)


# device time: 41778 ns/iter; 3.0419x vs baseline; 1.0075x over previous
import jax
import jax.numpy as jnp
from jax import lax
from jax.experimental import pallas as pl
from jax.experimental.pallas import tpu as pltpu

N_DEV = 16
B = 2
SQ = 128
SKV = 128
HQ = 4
DH = 64
BLK = 64
D_MODEL = 512
D_QK = HQ * DH
D_KV2 = 2 * D_QK
SKV_ALL = N_DEV * SKV


def kernel(x, Wq, K_ext, V_ext, Wo):
    c = jnp.concatenate(
        [K_ext.reshape(B, SKV, D_QK), V_ext.reshape(B, SKV, D_QK)],
        axis=-1).astype(jnp.bfloat16)

    def body(x_ref, wq_ref, c_ref, wo_ref, out_ref,
             g_ref, send_sems, recv_sems):
        my = lax.axis_index("i")

        def needs(s, d):
            t = (2 * d + 2 * s) % 3
            n0 = (t != 1) | (s == 0)
            n1 = t != 0
            return n0, n1

        g_ref[pl.ds(my, 1)] = c_ref[...][None]

        barrier = pltpu.get_barrier_semaphore()
        for o in range(1, N_DEV):
            pl.semaphore_signal(barrier, inc=1, device_id=((my + o) % N_DEV,),
                                device_id_type=pl.DeviceIdType.MESH)
        pl.semaphore_wait(barrier, N_DEV - 1)

        def for_each_send(action):
            for o in range(1, N_DEV):
                d = (my + o) % N_DEV
                n0, n1 = needs(my, d)
                roff = jnp.where(n0, 0, BLK)

                @pl.when(n0 & n1)
                def _(d=d, o=o):
                    action(pltpu.make_async_remote_copy(
                        src_ref=c_ref, dst_ref=g_ref.at[my],
                        send_sem=send_sems.at[o - 1],
                        recv_sem=recv_sems.at[o - 1],
                        device_id=(d,),
                        device_id_type=pl.DeviceIdType.MESH))

                @pl.when(n0 != n1)
                def _(d=d, o=o, roff=roff):
                    action(pltpu.make_async_remote_copy(
                        src_ref=c_ref.at[:, pl.ds(roff, BLK), :],
                        dst_ref=g_ref.at[my, :, pl.ds(roff, BLK), :],
                        send_sem=send_sems.at[o - 1],
                        recv_sem=recv_sems.at[o - 1],
                        device_id=(d,),
                        device_id_type=pl.DeviceIdType.MESH))

        for_each_send(lambda r: r.start())

        for o in range(1, N_DEV):
            s = (my - o) % N_DEV
            n0, n1 = needs(s, my)

            @pl.when(n0 != n1)
            def _(s=s, n0=n0):
                goff = jnp.where(n0, BLK, 0)
                g_ref[pl.ds(s, 1), :, pl.ds(goff, BLK),
                      pl.ds(D_QK, D_QK)] = jnp.zeros(
                          (1, B, BLK, D_QK), jnp.bfloat16)

        i_idx = lax.broadcasted_iota(jnp.int32, (SQ, SKV_ALL), 0)
        j_idx = lax.broadcasted_iota(jnp.int32, (SQ, SKV_ALL), 1)
        qb = (my * SQ + i_idx) // BLK
        kb = j_idx // BLK
        vis = (qb == kb) | (kb == 0) | ((qb + kb) % 3 == 0)
        q_all = [jnp.dot(x_ref[b], wq_ref[...],
                         preferred_element_type=jnp.float32)
                 for b in range(B)]

        for o in range(1, N_DEV):
            s = (my - o) % N_DEV
            n0, n1 = needs(s, my)
            roff = jnp.where(n0, 0, BLK)

            @pl.when(n0 & n1)
            def _(s=s, o=o):
                pltpu.make_async_remote_copy(
                    src_ref=c_ref, dst_ref=g_ref.at[s],
                    send_sem=send_sems.at[o - 1],
                    recv_sem=recv_sems.at[o - 1],
                    device_id=(s,),
                    device_id_type=pl.DeviceIdType.MESH).wait_recv()

            @pl.when(n0 != n1)
            def _(s=s, o=o, roff=roff):
                pltpu.make_async_remote_copy(
                    src_ref=c_ref.at[:, pl.ds(roff, BLK), :],
                    dst_ref=g_ref.at[s, :, pl.ds(roff, BLK), :],
                    send_sem=send_sems.at[o - 1],
                    recv_sem=recv_sems.at[o - 1],
                    device_id=(s,),
                    device_id_type=pl.DeviceIdType.MESH).wait_recv()

        for b in range(B):
            k_all = g_ref[:, b, :, 0:D_QK].reshape(SKV_ALL, D_QK)
            v_all = g_ref[:, b, :, D_QK:D_KV2].reshape(SKV_ALL, D_QK)
            ctxs = []
            for hh in range(HQ):
                q_h = q_all[b][:, hh * DH:(hh + 1) * DH].astype(jnp.bfloat16)
                k_h = k_all[:, hh * DH:(hh + 1) * DH]
                v_h = v_all[:, hh * DH:(hh + 1) * DH]
                s_ = lax.dot_general(
                    q_h, k_h, (((1,), (1,)), ((), ())),
                    preferred_element_type=jnp.float32) * 0.125
                s_ = jnp.where(vis, s_, -1e9)
                m = jnp.max(s_, axis=1, keepdims=True)
                w = jnp.exp(s_ - m)
                w = (w / jnp.sum(w, axis=1, keepdims=True)).astype(jnp.bfloat16)
                ctxs.append(jnp.dot(w, v_h,
                                    preferred_element_type=jnp.float32))
            ctx = jnp.concatenate(ctxs, axis=1)
            out_ref[b] = jnp.dot(ctx, wo_ref[...],
                                 preferred_element_type=jnp.float32)

        for_each_send(lambda r: r.wait_send())

    return pl.pallas_call(
        body,
        out_shape=jax.ShapeDtypeStruct((B, SQ, D_MODEL), jnp.float32),
        in_specs=[pl.BlockSpec(memory_space=pltpu.VMEM)] * 4,
        out_specs=pl.BlockSpec(memory_space=pltpu.VMEM),
        scratch_shapes=[
            pltpu.VMEM((N_DEV, B, SKV, D_KV2), jnp.bfloat16),
            pltpu.SemaphoreType.DMA((N_DEV - 1,)),
            pltpu.SemaphoreType.DMA((N_DEV - 1,)),
        ],
        compiler_params=pltpu.CompilerParams(collective_id=0),
    )(x, Wq, c, Wo)
